# baseline (speedup 1.0000x reference)
import jax
import jax.numpy as jnp
from jax import lax
from jax.experimental import pallas as pl
from jax.experimental.pallas import tpu as pltpu
from jax.experimental.pallas import tpu_sc as plsc

D = 32
NC = 2
NS = 16
NW = NC * NS
CHUNK = 128
DEPTH = 6


def _make_tca_body(n_u, blk):
    def _tca_body(xcat_ref, w3_ref, bcat_ref, xr_ref):
        row = (pl.program_id(0) * blk
               + jax.lax.broadcasted_iota(jnp.int32, (blk, 1), 0))
        b = jnp.where(row < n_u, bcat_ref[0], bcat_ref[1])
        x0 = jnp.dot(xcat_ref[...], w3_ref[...],
                     preferred_element_type=jnp.float32) + b
        xr_ref[...] = jnp.maximum(x0, 0.0)
    return _tca_body


def _mean_agg(p0_ref, p1_ref, d0_ref, d1_ref, w1_ref, b1_ref):
    st = p0_ref[0] + p1_ref[0]
    deg = jnp.maximum(d0_ref[0, :, 0:1] + d1_ref[0, :, 0:1], 1.0)
    agg = (jnp.dot(st[:, D:], w1_ref[...], preferred_element_type=jnp.float32)
           + jnp.dot(st[:, :D], b1_ref[...],
                     preferred_element_type=jnp.float32))
    return agg / deg


def _tcb_body(p0_ref, p1_ref, d0_ref, d1_ref, xprev_ref, root_ref, bias_ref,
              w1_ref, b1_ref, xr_ref):
    agg = _mean_agg(p0_ref, p1_ref, d0_ref, d1_ref, w1_ref, b1_ref)
    x1 = agg + jnp.dot(xprev_ref[...], root_ref[...],
                       preferred_element_type=jnp.float32) + bias_ref[...]
    xr_ref[...] = jnp.maximum(x1, 0.0)


def _tcc_body(p0_ref, p1_ref, d0_ref, d1_ref, xprev_ref, root_ref, bias_ref,
              w1_ref, b1_ref, out_ref):
    agg = _mean_agg(p0_ref, p1_ref, d0_ref, d1_ref, w1_ref, b1_ref)
    out_ref[...] = agg + jnp.dot(xprev_ref[...], root_ref[...],
                                 preferred_element_type=jnp.float32) + bias_ref[...]


def _tca(xcat, w3, bcat2, n_u):
    n = xcat.shape[0]
    blk = n // 5
    return pl.pallas_call(
        _make_tca_body(n_u, blk),
        grid=(5,),
        in_specs=[
            pl.BlockSpec((blk, 8), lambda i: (i, 0)),
            pl.BlockSpec((8, D), lambda i: (0, 0)),
            pl.BlockSpec((2, 1, D), lambda i: (0, 0, 0)),
        ],
        out_specs=pl.BlockSpec((blk, D), lambda i: (i, 0)),
        out_shape=jax.ShapeDtypeStruct((n, D), jnp.float32),
    )(xcat, w3, bcat2)


def _tc_layer(body, aggf, degf, xprev, root, bias2d, w1, b1):
    n = xprev.shape[0]
    blk = n // 5
    specs = [
        pl.BlockSpec((1, blk, 2 * D), lambda i: (0, i, 0)),
        pl.BlockSpec((1, blk, 2 * D), lambda i: (1, i, 0)),
        pl.BlockSpec((1, blk, 16), lambda i: (0, i, 0)),
        pl.BlockSpec((1, blk, 16), lambda i: (1, i, 0)),
        pl.BlockSpec((blk, D), lambda i: (i, 0)),
        pl.BlockSpec((D, D), lambda i: (0, 0)),
        pl.BlockSpec((1, D), lambda i: (0, 0)),
        pl.BlockSpec((D, D), lambda i: (0, 0)),
        pl.BlockSpec((D, D), lambda i: (0, 0)),
    ]
    return pl.pallas_call(
        body,
        grid=(5,),
        in_specs=specs,
        out_specs=pl.BlockSpec((blk, D), lambda i: (i, 0)),
        out_shape=jax.ShapeDtypeStruct((n, D), jnp.float32),
    )(aggf, aggf, degf, degf, xprev, root, bias2d, w1, b1)


def _sc_edge_call(xr, srcw, dstw, attrw, z64, z16, o16, nagg, nch, with_deg):
    rps = nagg // NS
    nblk = rps // CHUNK
    mesh = plsc.VectorSubcoreMesh(core_axis_name="c", subcore_axis_name="s")
    out_type = [jax.ShapeDtypeStruct((NC, nagg, 2 * D), jnp.float32)]
    scratch = [
        pltpu.VMEM((nch, CHUNK), jnp.int32),
        pltpu.VMEM((nch, CHUNK), jnp.int32),
        pltpu.VMEM((DEPTH, CHUNK), jnp.float32),
        pltpu.VMEM((DEPTH, CHUNK, D), jnp.float32),
        pltpu.VMEM((2, CHUNK, 2 * D), jnp.float32),
        pltpu.VMEM((CHUNK, 16), jnp.float32),
        pltpu.VMEM_SHARED((nagg, 2 * D), jnp.float32),
        pltpu.VMEM_SHARED((nagg, D), jnp.float32),
        pltpu.SemaphoreType.DMA((DEPTH,)),
        pltpu.SemaphoreType.DMA((DEPTH,)),
        pltpu.SemaphoreType.DMA((2,)),
        pltpu.SemaphoreType.DMA((2,)),
    ]
    if with_deg:
        out_type.append(jax.ShapeDtypeStruct((NC, nagg, 16), jnp.float32))
        scratch.append(pltpu.VMEM_SHARED((nagg, 16), jnp.float32))

    def body(xr_hbm, src_hbm, dst_hbm, attr_hbm, z64_hbm, z16_hbm, o16_hbm,
             *refs):
        if with_deg:
            (agg_out, deg_out, srcv, dstv, attrv, rowsv, msgv, onesv, aggS,
             tabS, sem_a, sem_g, sem_s, sem_d, degS) = refs
        else:
            (agg_out, srcv, dstv, attrv, rowsv, msgv, onesv, aggS,
             tabS, sem_a, sem_g, sem_s, sem_d) = refs
        c = lax.axis_index("c")
        s = lax.axis_index("s")
        w = c * NS + s

        pltpu.sync_copy(src_hbm.at[w], srcv)
        pltpu.sync_copy(dst_hbm.at[w], dstv)
        if with_deg:
            pltpu.sync_copy(o16_hbm, onesv)

        def start(j, b):
            pltpu.async_copy(attr_hbm.at[w * nch + j], attrv.at[b],
                             sem_a.at[b])

            @pl.when(lax.rem(j, 2) == 0)
            def _():
                pltpu.async_copy(tabS.at[srcv.at[j]], rowsv.at[b],
                                 sem_g.at[b])

            @pl.when(lax.rem(j, 2) == 1)
            def _():
                pltpu.async_copy(xr_hbm.at[srcv.at[j]], rowsv.at[b],
                                 sem_g.at[b])

        rtab = xr_hbm.shape[0] // NS
        pltpu.sync_copy(xr_hbm.at[pl.ds(s * rtab, rtab)],
                        tabS.at[pl.ds(s * rtab, rtab)])

        @pl.loop(0, nblk)
        def _(t):
            base = s * rps + t * CHUNK
            pltpu.sync_copy(z64_hbm, aggS.at[pl.ds(base, CHUNK)])
            if with_deg:
                pltpu.sync_copy(z16_hbm, degS.at[pl.ds(base, CHUNK)])

        plsc.subcore_barrier()

        for b in range(DEPTH):
            start(b, b)

        @pl.loop(0, nch)
        def _(j):
            b = lax.rem(j, DEPTH)
            mb = lax.rem(j, 2)
            pltpu.make_async_copy(attr_hbm.at[0], attrv.at[b],
                                  sem_a.at[b]).wait()
            pltpu.make_async_copy(xr_hbm.at[srcv.at[0]], rowsv.at[b],
                                  sem_g.at[b]).wait()
            rv = rowsv.at[b]
            av_ = attrv.at[b]
            mv = msgv.at[mb]

            @pl.when(j >= 2)
            def _():
                pltpu.make_async_copy(msgv.at[mb], aggS.at[dstv.at[0]],
                                      sem_s.at[mb]).wait()
                if with_deg:
                    pltpu.make_async_copy(onesv, degS.at[dstv.at[0]],
                                          sem_d.at[mb]).wait()

            @pl.loop(0, CHUNK, step=4)
            def _(k):
                for u in range(4):
                    ku = k + u
                    av = plsc.load_gather(av_, [jnp.full((16,), ku,
                                                         jnp.int32)])
                    xa = rv[ku, 0:16]
                    xb = rv[ku, 16:32]
                    mv[ku, 0:16] = xa
                    mv[ku, 16:32] = xb
                    mv[ku, 32:48] = av * xa
                    mv[ku, 48:64] = av * xb

            @pl.when(j + DEPTH < nch)
            def _():
                start(j + DEPTH, b)

            pltpu.async_copy(msgv.at[mb], aggS.at[dstv.at[j]], sem_s.at[mb],
                             add=True)
            if with_deg:
                pltpu.async_copy(onesv, degS.at[dstv.at[j]], sem_d.at[mb],
                                 add=True)

        for mb in range(2):
            pltpu.make_async_copy(msgv.at[mb], aggS.at[dstv.at[0]],
                                  sem_s.at[mb]).wait()
            if with_deg:
                pltpu.make_async_copy(onesv, degS.at[dstv.at[0]],
                                      sem_d.at[mb]).wait()

        plsc.subcore_barrier()

        @pl.loop(0, nblk)
        def _(t):
            base = s * rps + t * CHUNK
            pltpu.sync_copy(aggS.at[pl.ds(base, CHUNK)],
                            agg_out.at[c].at[pl.ds(base, CHUNK)])
            if with_deg:
                pltpu.sync_copy(degS.at[pl.ds(base, CHUNK)],
                                deg_out.at[c].at[pl.ds(base, CHUNK)])

    fn = pl.kernel(
        body, mesh=mesh, out_type=out_type, scratch_types=scratch,
        compiler_params=pltpu.CompilerParams(use_tc_tiling_on_sc=False,
                                             needs_layout_passes=False,
                                             disable_bounds_checks=True))
    return fn(xr, srcw, dstw, attrw, z64, z16, o16)


def kernel(x_u, x_v, edge_index, edge_attribute, i, dummy,
           Wu, bu, Wv, bv, Wl1, bl1, root, bias):
    n_u = x_u.shape[0]
    n_v = x_v.shape[0]
    n = n_u + n_v
    e = edge_index.shape[1]

    w1 = Wl1.reshape(D, D)
    b1 = bl1.reshape(D, D)
    w3 = jnp.concatenate([Wu, Wv, jnp.zeros((5, D), jnp.float32)], axis=0)
    bcat2 = jnp.stack([bu, bv], axis=0).reshape(2, 1, D)
    xcat = jnp.concatenate([
        jnp.pad(x_u, ((0, 0), (0, 7))),
        jnp.pad(x_v, ((0, 0), (1, 5))),
    ], axis=0)
    bias2d = bias.reshape(1, D)

    epad = -(-e // (NW * CHUNK)) * (NW * CHUNK)
    nch = epad // (NW * CHUNK)
    pad = epad - e
    nagg = -(-(n + 1) // (NS * CHUNK)) * (NS * CHUNK)
    pad_src = jnp.arange(pad, dtype=jnp.int32) % n
    pad_dst = n + jnp.arange(pad, dtype=jnp.int32) % (nagg - n)
    src = jnp.concatenate([edge_index[0], pad_src])
    dst = jnp.concatenate([edge_index[1], pad_dst])
    attr = jnp.concatenate([edge_attribute[:, 0],
                            jnp.zeros((pad,), jnp.float32)])
    srcw = src.reshape(NW, nch, CHUNK)
    dstw = dst.reshape(NW, nch, CHUNK)
    attrw = attr.reshape(NW * nch, CHUNK)
    z64 = jnp.zeros((CHUNK, 2 * D), jnp.float32)
    z16 = jnp.zeros((CHUNK, 16), jnp.float32)
    o16 = jnp.ones((CHUNK, 16), jnp.float32)

    xr0 = _tca(xcat, w3, bcat2, n_u)
    aggf0, degf = _sc_edge_call(xr0, srcw, dstw, attrw, z64, z16, o16,
                                nagg, nch, True)
    xr1 = _tc_layer(_tcb_body, aggf0, degf, xr0, root, bias2d, w1, b1)
    aggf1 = _sc_edge_call(xr1, srcw, dstw, attrw, z64, z16, o16,
                          nagg, nch, False)[0]
    return _tc_layer(_tcc_body, aggf1, degf, xr1, root, bias2d, w1, b1)


# speedup vs baseline: 1.1103x; 1.1103x over previous
import jax
import jax.numpy as jnp
from jax import lax
from jax.experimental import pallas as pl
from jax.experimental.pallas import tpu as pltpu
from jax.experimental.pallas import tpu_sc as plsc

D = 32
NC = 2
NS = 16
NW = NC * NS
CHUNK = 128
DEPTH = 6


def _make_tca_body(n_u, blk):
    def _tca_body(xcat_ref, w3_ref, bcat_ref, xr_ref):
        row = (pl.program_id(0) * blk
               + jax.lax.broadcasted_iota(jnp.int32, (blk, 1), 0))
        b = jnp.where(row < n_u, bcat_ref[0], bcat_ref[1])
        x0 = jnp.dot(xcat_ref[...], w3_ref[...],
                     preferred_element_type=jnp.float32) + b
        xr_ref[...] = jnp.maximum(x0, 0.0)
    return _tca_body


def _mean_agg(p0_ref, p1_ref, d0_ref, d1_ref, w1_ref, b1_ref):
    st = p0_ref[0] + p1_ref[0]
    deg = jnp.maximum(d0_ref[0, :, 0:1] + d1_ref[0, :, 0:1], 1.0)
    agg = (jnp.dot(st[:, D:], w1_ref[...], preferred_element_type=jnp.float32)
           + jnp.dot(st[:, :D], b1_ref[...],
                     preferred_element_type=jnp.float32))
    return agg / deg


def _tcb_body(p0_ref, p1_ref, d0_ref, d1_ref, xprev_ref, root_ref, bias_ref,
              w1_ref, b1_ref, xr_ref):
    agg = _mean_agg(p0_ref, p1_ref, d0_ref, d1_ref, w1_ref, b1_ref)
    x1 = agg + jnp.dot(xprev_ref[...], root_ref[...],
                       preferred_element_type=jnp.float32) + bias_ref[...]
    xr_ref[...] = jnp.maximum(x1, 0.0)


def _tcc_body(p0_ref, p1_ref, d0_ref, d1_ref, xprev_ref, root_ref, bias_ref,
              w1_ref, b1_ref, out_ref):
    agg = _mean_agg(p0_ref, p1_ref, d0_ref, d1_ref, w1_ref, b1_ref)
    out_ref[...] = agg + jnp.dot(xprev_ref[...], root_ref[...],
                                 preferred_element_type=jnp.float32) + bias_ref[...]


def _tca(xcat, w3, bcat2, n_u):
    n = xcat.shape[0]
    blk = n // 5
    return pl.pallas_call(
        _make_tca_body(n_u, blk),
        grid=(5,),
        in_specs=[
            pl.BlockSpec((blk, 8), lambda i: (i, 0)),
            pl.BlockSpec((8, D), lambda i: (0, 0)),
            pl.BlockSpec((2, 1, D), lambda i: (0, 0, 0)),
        ],
        out_specs=pl.BlockSpec((blk, D), lambda i: (i, 0)),
        out_shape=jax.ShapeDtypeStruct((n, D), jnp.float32),
    )(xcat, w3, bcat2)


def _tc_layer(body, aggf, degf, xprev, root, bias2d, w1, b1):
    n = xprev.shape[0]
    blk = n // 5
    specs = [
        pl.BlockSpec((1, blk, 2 * D), lambda i: (0, i, 0)),
        pl.BlockSpec((1, blk, 2 * D), lambda i: (1, i, 0)),
        pl.BlockSpec((1, blk, 16), lambda i: (0, i, 0)),
        pl.BlockSpec((1, blk, 16), lambda i: (1, i, 0)),
        pl.BlockSpec((blk, D), lambda i: (i, 0)),
        pl.BlockSpec((D, D), lambda i: (0, 0)),
        pl.BlockSpec((1, D), lambda i: (0, 0)),
        pl.BlockSpec((D, D), lambda i: (0, 0)),
        pl.BlockSpec((D, D), lambda i: (0, 0)),
    ]
    return pl.pallas_call(
        body,
        grid=(5,),
        in_specs=specs,
        out_specs=pl.BlockSpec((blk, D), lambda i: (i, 0)),
        out_shape=jax.ShapeDtypeStruct((n, D), jnp.float32),
    )(aggf, aggf, degf, degf, xprev, root, bias2d, w1, b1)


def _sc_edge_call(xr, srcw, dstw, attrw, z64, z16, o16, nagg, nch, with_deg):
    rps = nagg // NS
    nblk = rps // CHUNK
    mesh = plsc.VectorSubcoreMesh(core_axis_name="c", subcore_axis_name="s")
    out_type = [jax.ShapeDtypeStruct((NC, nagg, 2 * D), jnp.float32)]
    scratch = [
        pltpu.VMEM((nch, CHUNK), jnp.int32),
        pltpu.VMEM((nch, CHUNK), jnp.int32),
        pltpu.VMEM((DEPTH, CHUNK), jnp.float32),
        pltpu.VMEM((DEPTH, CHUNK, D), jnp.float32),
        pltpu.VMEM((2, CHUNK, 2 * D), jnp.float32),
        pltpu.VMEM((CHUNK, 16), jnp.float32),
        pltpu.VMEM_SHARED((nagg, 2 * D), jnp.float32),
        pltpu.SemaphoreType.DMA((DEPTH,)),
        pltpu.SemaphoreType.DMA((DEPTH,)),
        pltpu.SemaphoreType.DMA((2,)),
        pltpu.SemaphoreType.DMA((2,)),
    ]
    if with_deg:
        out_type.append(jax.ShapeDtypeStruct((NC, nagg, 16), jnp.float32))
        scratch.append(pltpu.VMEM_SHARED((nagg, 16), jnp.float32))

    def body(xr_hbm, src_hbm, dst_hbm, attr_hbm, z64_hbm, z16_hbm, o16_hbm,
             *refs):
        if with_deg:
            (agg_out, deg_out, srcv, dstv, attrv, rowsv, msgv, onesv, aggS,
             sem_a, sem_g, sem_s, sem_d, degS) = refs
        else:
            (agg_out, srcv, dstv, attrv, rowsv, msgv, onesv, aggS,
             sem_a, sem_g, sem_s, sem_d) = refs
        c = lax.axis_index("c")
        s = lax.axis_index("s")
        w = c * NS + s

        pltpu.sync_copy(src_hbm.at[w], srcv)
        pltpu.sync_copy(dst_hbm.at[w], dstv)
        if with_deg:
            pltpu.sync_copy(o16_hbm, onesv)

        def start(j, b):
            pltpu.async_copy(attr_hbm.at[w * nch + j], attrv.at[b],
                             sem_a.at[b])
            pltpu.async_copy(xr_hbm.at[srcv.at[j]], rowsv.at[b],
                             sem_g.at[b])

        @pl.loop(0, nblk)
        def _(t):
            base = s * rps + t * CHUNK
            pltpu.sync_copy(z64_hbm, aggS.at[pl.ds(base, CHUNK)])
            if with_deg:
                pltpu.sync_copy(z16_hbm, degS.at[pl.ds(base, CHUNK)])

        plsc.subcore_barrier()

        for b in range(DEPTH):
            start(b, b)

        @pl.loop(0, nch)
        def _(j):
            b = lax.rem(j, DEPTH)
            mb = lax.rem(j, 2)
            pltpu.make_async_copy(attr_hbm.at[0], attrv.at[b],
                                  sem_a.at[b]).wait()
            pltpu.make_async_copy(xr_hbm.at[srcv.at[0]], rowsv.at[b],
                                  sem_g.at[b]).wait()
            rv = rowsv.at[b]
            av_ = attrv.at[b]
            mv = msgv.at[mb]

            @pl.when(j >= 2)
            def _():
                pltpu.make_async_copy(msgv.at[mb], aggS.at[dstv.at[0]],
                                      sem_s.at[mb]).wait()
                if with_deg:
                    pltpu.make_async_copy(onesv, degS.at[dstv.at[0]],
                                          sem_d.at[mb]).wait()

            @plsc.parallel_loop(0, CHUNK, unroll=8)
            def _(ku):
                av = plsc.load_gather(av_, [jnp.full((16,), ku, jnp.int32)])
                xa = rv[ku, 0:16]
                xb = rv[ku, 16:32]
                mv[ku, 0:16] = xa
                mv[ku, 16:32] = xb
                mv[ku, 32:48] = av * xa
                mv[ku, 48:64] = av * xb

            @pl.when(j + DEPTH < nch)
            def _():
                start(j + DEPTH, b)

            pltpu.async_copy(msgv.at[mb], aggS.at[dstv.at[j]], sem_s.at[mb],
                             add=True)
            if with_deg:
                pltpu.async_copy(onesv, degS.at[dstv.at[j]], sem_d.at[mb],
                                 add=True)

        for mb in range(2):
            pltpu.make_async_copy(msgv.at[mb], aggS.at[dstv.at[0]],
                                  sem_s.at[mb]).wait()
            if with_deg:
                pltpu.make_async_copy(onesv, degS.at[dstv.at[0]],
                                      sem_d.at[mb]).wait()

        plsc.subcore_barrier()

        @pl.loop(0, nblk)
        def _(t):
            base = s * rps + t * CHUNK
            pltpu.sync_copy(aggS.at[pl.ds(base, CHUNK)],
                            agg_out.at[c].at[pl.ds(base, CHUNK)])
            if with_deg:
                pltpu.sync_copy(degS.at[pl.ds(base, CHUNK)],
                                deg_out.at[c].at[pl.ds(base, CHUNK)])

    fn = pl.kernel(
        body, mesh=mesh, out_type=out_type, scratch_types=scratch,
        compiler_params=pltpu.CompilerParams(use_tc_tiling_on_sc=False,
                                             needs_layout_passes=False,
                                             disable_bounds_checks=True))
    return fn(xr, srcw, dstw, attrw, z64, z16, o16)


def kernel(x_u, x_v, edge_index, edge_attribute, i, dummy,
           Wu, bu, Wv, bv, Wl1, bl1, root, bias):
    n_u = x_u.shape[0]
    n_v = x_v.shape[0]
    n = n_u + n_v
    e = edge_index.shape[1]

    w1 = Wl1.reshape(D, D)
    b1 = bl1.reshape(D, D)
    w3 = jnp.concatenate([Wu, Wv, jnp.zeros((5, D), jnp.float32)], axis=0)
    bcat2 = jnp.stack([bu, bv], axis=0).reshape(2, 1, D)
    xcat = jnp.concatenate([
        jnp.pad(x_u, ((0, 0), (0, 7))),
        jnp.pad(x_v, ((0, 0), (1, 5))),
    ], axis=0)
    bias2d = bias.reshape(1, D)

    epad = -(-e // (NW * CHUNK)) * (NW * CHUNK)
    nch = epad // (NW * CHUNK)
    pad = epad - e
    nagg = -(-(n + 1) // (NS * CHUNK)) * (NS * CHUNK)
    pad_src = jnp.arange(pad, dtype=jnp.int32) % n
    pad_dst = n + jnp.arange(pad, dtype=jnp.int32) % (nagg - n)
    src = jnp.concatenate([edge_index[0], pad_src])
    dst = jnp.concatenate([edge_index[1], pad_dst])
    attr = jnp.concatenate([edge_attribute[:, 0],
                            jnp.zeros((pad,), jnp.float32)])
    srcw = src.reshape(NW, nch, CHUNK)
    dstw = dst.reshape(NW, nch, CHUNK)
    attrw = attr.reshape(NW * nch, CHUNK)
    z64 = jnp.zeros((CHUNK, 2 * D), jnp.float32)
    z16 = jnp.zeros((CHUNK, 16), jnp.float32)
    o16 = jnp.ones((CHUNK, 16), jnp.float32)

    xr0 = _tca(xcat, w3, bcat2, n_u)
    aggf0, degf = _sc_edge_call(xr0, srcw, dstw, attrw, z64, z16, o16,
                                nagg, nch, True)
    xr1 = _tc_layer(_tcb_body, aggf0, degf, xr0, root, bias2d, w1, b1)
    aggf1 = _sc_edge_call(xr1, srcw, dstw, attrw, z64, z16, o16,
                          nagg, nch, False)[0]
    return _tc_layer(_tcc_body, aggf1, degf, xr1, root, bias2d, w1, b1)


# speedup vs baseline: 1.1898x; 1.0716x over previous
import jax
import jax.numpy as jnp
from jax import lax
from jax.experimental import pallas as pl
from jax.experimental.pallas import tpu as pltpu
from jax.experimental.pallas import tpu_sc as plsc

D = 32
NC = 2
NS = 16
NW = NC * NS
CHUNK = 128
DEPTH = 6


def _make_tca_body(n_u_q, blk):
    def _tca_body(xcatq_ref, w3q_ref, bq_ref, xrq_ref):
        row = (pl.program_id(0) * blk
               + jax.lax.broadcasted_iota(jnp.int32, (blk, 1), 0))
        b = jnp.where(row < n_u_q, bq_ref[0], bq_ref[1])
        x0 = jnp.dot(xcatq_ref[...], w3q_ref[...],
                     preferred_element_type=jnp.float32) + b
        xrq_ref[...] = jnp.maximum(x0, 0.0)
    return _tca_body


def _tcb_body(p0_ref, p1_ref, d0_ref, d1_ref, xprev_ref, rootq_ref,
              biasq_ref, wq_ref, xr_ref):
    st = p0_ref[0] + p1_ref[0]
    deg = jnp.maximum(d0_ref[0] + d1_ref[0], 1.0)
    agg = jnp.dot(st, wq_ref[...], preferred_element_type=jnp.float32) / deg
    x1 = agg + jnp.dot(xprev_ref[...], rootq_ref[...],
                       preferred_element_type=jnp.float32) + biasq_ref[...]
    xr_ref[...] = jnp.maximum(x1, 0.0)


def _tcc_body(p0_ref, p1_ref, d0_ref, d1_ref, xprev_ref, rootq_ref,
              biasq_ref, wq_ref, out_ref):
    st = p0_ref[0] + p1_ref[0]
    deg = jnp.maximum(d0_ref[0] + d1_ref[0], 1.0)
    agg = jnp.dot(st, wq_ref[...], preferred_element_type=jnp.float32) / deg
    out_ref[...] = agg + jnp.dot(xprev_ref[...], rootq_ref[...],
                                 preferred_element_type=jnp.float32) + biasq_ref[...]


def _tca(xcatq, w3q, bq, n_u_q):
    nq = xcatq.shape[0]
    blk = nq // 5
    return pl.pallas_call(
        _make_tca_body(n_u_q, blk),
        grid=(5,),
        in_specs=[
            pl.BlockSpec((blk, 4 * 8), lambda i: (i, 0)),
            pl.BlockSpec((4 * 8, 4 * D), lambda i: (0, 0)),
            pl.BlockSpec((2, 1, 4 * D), lambda i: (0, 0, 0)),
        ],
        out_specs=pl.BlockSpec((blk, 4 * D), lambda i: (i, 0)),
        out_shape=jax.ShapeDtypeStruct((nq, 4 * D), jnp.float32),
    )(xcatq, w3q, bq)


def _tc_layer(body, aggq, degq, xprevq, rootq, biasq, wq):
    nq = xprevq.shape[0]
    blk = nq // 5
    specs = [
        pl.BlockSpec((1, blk, 8 * D), lambda i: (0, i, 0)),
        pl.BlockSpec((1, blk, 8 * D), lambda i: (1, i, 0)),
        pl.BlockSpec((1, blk, 4 * D), lambda i: (0, i, 0)),
        pl.BlockSpec((1, blk, 4 * D), lambda i: (1, i, 0)),
        pl.BlockSpec((blk, 4 * D), lambda i: (i, 0)),
        pl.BlockSpec((4 * D, 4 * D), lambda i: (0, 0)),
        pl.BlockSpec((1, 4 * D), lambda i: (0, 0)),
        pl.BlockSpec((8 * D, 4 * D), lambda i: (0, 0)),
    ]
    return pl.pallas_call(
        body,
        grid=(5,),
        in_specs=specs,
        out_specs=pl.BlockSpec((blk, 4 * D), lambda i: (i, 0)),
        out_shape=jax.ShapeDtypeStruct((nq, 4 * D), jnp.float32),
    )(aggq, aggq, degq, degq, xprevq, rootq, biasq, wq)


def _sc_edge_call(xr, srcw, dstw, attrw, z64, z32, o32, nagg, nch, with_deg):
    rps = nagg // NS
    nblk = rps // CHUNK
    mesh = plsc.VectorSubcoreMesh(core_axis_name="c", subcore_axis_name="s")
    out_type = [jax.ShapeDtypeStruct((NC, nagg, 2 * D), jnp.float32)]
    scratch = [
        pltpu.VMEM((nch, CHUNK), jnp.int32),
        pltpu.VMEM((nch, CHUNK), jnp.int32),
        pltpu.VMEM((DEPTH, CHUNK), jnp.float32),
        pltpu.VMEM((DEPTH, CHUNK, D), jnp.float32),
        pltpu.VMEM((2, CHUNK, 2 * D), jnp.float32),
        pltpu.VMEM((CHUNK, D), jnp.float32),
        pltpu.VMEM_SHARED((nagg, 2 * D), jnp.float32),
        pltpu.SemaphoreType.DMA((DEPTH,)),
        pltpu.SemaphoreType.DMA((DEPTH,)),
        pltpu.SemaphoreType.DMA((2,)),
        pltpu.SemaphoreType.DMA((2,)),
    ]
    if with_deg:
        out_type.append(jax.ShapeDtypeStruct((NC, nagg, D),
                                              jnp.float32))
        scratch.append(pltpu.VMEM_SHARED((nagg, D), jnp.float32))

    def body(xr_hbm, src_hbm, dst_hbm, attr_hbm, z64_hbm, z32_hbm,
             o32_hbm, *refs):
        if with_deg:
            (agg_out, deg_out, srcv, dstv, attrv, rowsv, msgv, onesv, aggS,
             sem_a, sem_g, sem_s, sem_d, degS) = refs
        else:
            (agg_out, srcv, dstv, attrv, rowsv, msgv, onesv, aggS,
             sem_a, sem_g, sem_s, sem_d) = refs
        c = lax.axis_index("c")
        s = lax.axis_index("s")
        w = c * NS + s

        pltpu.sync_copy(src_hbm.at[w], srcv)
        pltpu.sync_copy(dst_hbm.at[w], dstv)
        if with_deg:
            pltpu.sync_copy(o32_hbm, onesv)

        def start(j, b):
            pltpu.async_copy(attr_hbm.at[w * nch + j], attrv.at[b],
                             sem_a.at[b])
            pltpu.async_copy(xr_hbm.at[srcv.at[j]], rowsv.at[b],
                             sem_g.at[b])

        @pl.loop(0, nblk)
        def _(t):
            base = s * rps + t * CHUNK
            pltpu.sync_copy(z64_hbm, aggS.at[pl.ds(base, CHUNK)])
            if with_deg:
                pltpu.sync_copy(z32_hbm, degS.at[pl.ds(base, CHUNK)])

        plsc.subcore_barrier()

        for b in range(DEPTH):
            start(b, b)

        @pl.loop(0, nch)
        def _(j):
            b = lax.rem(j, DEPTH)
            mb = lax.rem(j, 2)
            pltpu.make_async_copy(attr_hbm.at[0], attrv.at[b],
                                  sem_a.at[b]).wait()
            pltpu.make_async_copy(xr_hbm.at[srcv.at[0]], rowsv.at[b],
                                  sem_g.at[b]).wait()
            rv = rowsv.at[b]
            av_ = attrv.at[b]
            mv = msgv.at[mb]

            @pl.when(j >= 2)
            def _():
                pltpu.make_async_copy(msgv.at[mb], aggS.at[dstv.at[0]],
                                      sem_s.at[mb]).wait()
                if with_deg:
                    pltpu.make_async_copy(onesv, degS.at[dstv.at[0]],
                                          sem_d.at[mb]).wait()

            @plsc.parallel_loop(0, CHUNK, unroll=8)
            def _(ku):
                av = plsc.load_gather(av_, [jnp.full((16,), ku, jnp.int32)])
                xa = rv[ku, 0:16]
                xb = rv[ku, 16:32]
                mv[ku, 0:16] = xa
                mv[ku, 16:32] = xb
                mv[ku, 32:48] = av * xa
                mv[ku, 48:64] = av * xb

            @pl.when(j + DEPTH < nch)
            def _():
                start(j + DEPTH, b)

            pltpu.async_copy(msgv.at[mb], aggS.at[dstv.at[j]], sem_s.at[mb],
                             add=True)
            if with_deg:
                pltpu.async_copy(onesv, degS.at[dstv.at[j]], sem_d.at[mb],
                                 add=True)

        for mb in range(2):
            pltpu.make_async_copy(msgv.at[mb], aggS.at[dstv.at[0]],
                                  sem_s.at[mb]).wait()
            if with_deg:
                pltpu.make_async_copy(onesv, degS.at[dstv.at[0]],
                                      sem_d.at[mb]).wait()

        plsc.subcore_barrier()

        @pl.loop(0, nblk)
        def _(t):
            base = s * rps + t * CHUNK
            pltpu.sync_copy(aggS.at[pl.ds(base, CHUNK)],
                            agg_out.at[c].at[pl.ds(base, CHUNK)])
            if with_deg:
                pltpu.sync_copy(degS.at[pl.ds(base, CHUNK)],
                                deg_out.at[c].at[pl.ds(base, CHUNK)])

    fn = pl.kernel(
        body, mesh=mesh, out_type=out_type, scratch_types=scratch,
        compiler_params=pltpu.CompilerParams(use_tc_tiling_on_sc=False,
                                             needs_layout_passes=False,
                                             disable_bounds_checks=True))
    return fn(xr, srcw, dstw, attrw, z64, z32, o32)


def kernel(x_u, x_v, edge_index, edge_attribute, i, dummy,
           Wu, bu, Wv, bv, Wl1, bl1, root, bias):
    n_u = x_u.shape[0]
    n_v = x_v.shape[0]
    n = n_u + n_v
    e = edge_index.shape[1]

    w1 = Wl1.reshape(D, D)
    b1 = bl1.reshape(D, D)
    eye4 = jnp.eye(4, dtype=jnp.float32)
    wq = jnp.kron(eye4, jnp.concatenate([b1, w1], axis=0))
    rootq = jnp.kron(eye4, root)
    w3q = jnp.kron(eye4, jnp.concatenate(
        [Wu, Wv, jnp.zeros((5, D), jnp.float32)], axis=0))
    biasq = jnp.tile(bias, 4).reshape(1, 4 * D)
    bq = jnp.stack([jnp.tile(bu, 4), jnp.tile(bv, 4)]).reshape(2, 1, 4 * D)

    epad = -(-e // (NW * CHUNK)) * (NW * CHUNK)
    nch = epad // (NW * CHUNK)
    pad = epad - e
    nagg = -(-(n + 1) // (NS * CHUNK)) * (NS * CHUNK)
    nq = nagg // 4

    xcatq = jnp.concatenate([
        jnp.pad(x_u, ((0, 0), (0, 7))).reshape(n_u // 4, 32),
        jnp.pad(x_v, ((0, 0), (1, 5))).reshape(n_v // 4, 32),
        jnp.zeros((nq - n // 4, 32), jnp.float32),
    ], axis=0)

    pad_src = jnp.arange(pad, dtype=jnp.int32) % n
    pad_dst = n + jnp.arange(pad, dtype=jnp.int32) % (nagg - n)
    src = jnp.concatenate([edge_index[0], pad_src])
    dst = jnp.concatenate([edge_index[1], pad_dst])
    attr = jnp.concatenate([edge_attribute[:, 0],
                            jnp.zeros((pad,), jnp.float32)])
    srcw = src.reshape(NW, nch, CHUNK)
    dstw = dst.reshape(NW, nch, CHUNK)
    attrw = attr.reshape(NW * nch, CHUNK)
    z64 = jnp.zeros((CHUNK, 2 * D), jnp.float32)
    z32 = jnp.zeros((CHUNK, D), jnp.float32)
    o32 = jnp.ones((CHUNK, D), jnp.float32)

    xrq0 = _tca(xcatq, w3q, bq, n_u // 4)
    aggf0, degf = _sc_edge_call(xrq0.reshape(nagg, D), srcw, dstw, attrw,
                                z64, z32, o32, nagg, nch, True)
    aggq0 = aggf0.reshape(NC, nq, 8 * D)
    degq = degf.reshape(NC, nq, 4 * D)
    xrq1 = _tc_layer(_tcb_body, aggq0, degq, xrq0, rootq, biasq, wq)
    aggf1 = _sc_edge_call(xrq1.reshape(nagg, D), srcw, dstw, attrw,
                          z64, z32, o32, nagg, nch, False)[0]
    aggq1 = aggf1.reshape(NC, nq, 8 * D)
    x2q = _tc_layer(_tcc_body, aggq1, degq, xrq1, rootq, biasq, wq)
    return x2q.reshape(nagg, D)[:n]
